# baseline (speedup 1.0000x reference)
import functools

import jax
import jax.numpy as jnp
from jax import lax
from jax.experimental import pallas as pl
from jax.experimental.pallas import tpu as pltpu
from jax.experimental.pallas import tpu_sc as plsc

_B, _L, _E, _W = 8, 2048, 768, 1024
_BL = _B * _L
_BW = _B * _W
_EC = 128
_NCH = 3
_RT = 1024
_SB = 128
_NSB = _RT // _SB
_RO = _BW // 16

_mesh = plsc.VectorSubcoreMesh(core_axis_name="c", subcore_axis_name="s")


@functools.partial(
    pl.kernel,
    out_type=jax.ShapeDtypeStruct((_BW, _E), jnp.float32),
    mesh=_mesh,
    scratch_types=[
        pltpu.VMEM((_NSB, _SB), jnp.int32),
        pltpu.VMEM((_RO + 16,), jnp.float32),
        pltpu.VMEM((_SB, _EC), jnp.float32),
        pltpu.VMEM((_SB, _EC), jnp.float32),
        pltpu.VMEM((_SB, _EC), jnp.float32),
        pltpu.VMEM_SHARED((_BW, _EC), jnp.float32),
    ],
)
def _sc_pool(seg_hbm, emb_hbm, out_hbm, fid2, invc, in_buf, ztab, ones_b,
             tab_sh):
    s = lax.axis_index("s")
    c = lax.axis_index("c")
    row0 = s * _RT
    bW = (s // 2) * _W
    own = s * _RO

    zero16 = jnp.zeros((16,), jnp.float32)
    one16 = jnp.ones((16,), jnp.float32)

    def _fill(r, carry):
        for j in range(_EC // 16):
            ztab[r, pl.ds(16 * j, 16)] = zero16
            ones_b[r, pl.ds(16 * j, 16)] = one16
        return carry

    lax.fori_loop(0, _SB, _fill, 0)

    pltpu.sync_copy(seg_hbm.at[pl.ds(s * _NSB, _NSB)], fid2)

    def _addb(r, carry):
        for j in range(_SB // 16):
            fid2[r, pl.ds(16 * j, 16)] = fid2[r, pl.ds(16 * j, 16)] + bW
        return carry

    lax.fori_loop(0, _NSB, _addb, 0)

    for i in range(_RO // _SB):
        pltpu.sync_copy(ztab, tab_sh.at[pl.ds(own + i * _SB, _SB)])
    plsc.subcore_barrier()

    for sb in range(_NSB):
        pltpu.sync_copy(ones_b, tab_sh.at[fid2.at[sb]], add=True)
    plsc.subcore_barrier()

    for i in range(_RO // _SB):
        pltpu.sync_copy(tab_sh.at[pl.ds(own + i * _SB, _SB)], in_buf)

        def _binv(r, carry, i=i):
            v = in_buf[r, pl.ds(0, 16)]
            invc[pl.ds(i * _SB + r, 16)] = 1.0 / jnp.maximum(v, 1.0)
            return carry

        lax.fori_loop(0, _SB, _binv, 0)
    for i in range(_RO // _SB):
        pltpu.sync_copy(ztab, tab_sh.at[pl.ds(own + i * _SB, _SB)])
    plsc.subcore_barrier()

    for k in range(_NCH):
        e0 = (c * _NCH + k) * _EC
        for sb in range(_NSB):
            pltpu.sync_copy(
                emb_hbm.at[pl.ds(row0 + sb * _SB, _SB), pl.ds(e0, _EC)],
                in_buf)
            pltpu.sync_copy(in_buf, tab_sh.at[fid2.at[sb]], add=True)
        plsc.subcore_barrier()
        for i in range(_RO // _SB):
            pltpu.sync_copy(tab_sh.at[pl.ds(own + i * _SB, _SB)], in_buf)

            def _scale(r, carry, i=i):
                cs = invc[pl.ds(i * _SB + r, 16)][0]
                for j in range(_EC // 16):
                    in_buf[r, pl.ds(16 * j, 16)] = (
                        in_buf[r, pl.ds(16 * j, 16)] * cs)
                return carry

            lax.fori_loop(0, _SB, _scale, 0)
            pltpu.sync_copy(
                in_buf,
                out_hbm.at[pl.ds(own + i * _SB, _SB), pl.ds(e0, _EC)])
        if k + 1 < _NCH:
            for i in range(_RO // _SB):
                pltpu.sync_copy(ztab, tab_sh.at[pl.ds(own + i * _SB, _SB)])
        plsc.subcore_barrier()


def kernel(subword_embs, segment_ids):
    seg2 = segment_ids.reshape(_SB, _SB).astype(jnp.int32)
    emb2 = subword_embs.reshape(_BL, _E)
    out = _sc_pool(seg2, emb2)
    return out.reshape(_B, _W, _E)


# speedup vs baseline: 1.2903x; 1.2903x over previous
import functools

import jax
import jax.numpy as jnp
from jax import lax
from jax.experimental import pallas as pl
from jax.experimental.pallas import tpu as pltpu
from jax.experimental.pallas import tpu_sc as plsc

_B, _L, _E, _W = 8, 2048, 768, 1024
_BL = _B * _L
_BW = _B * _W
_EC = 128
_NCH = 3
_RT = 1024
_SB = 128
_NSB = _RT // _SB
_RO = _BW // 16

_mesh = plsc.VectorSubcoreMesh(core_axis_name="c", subcore_axis_name="s")


@functools.partial(
    pl.kernel,
    out_type=jax.ShapeDtypeStruct((_BW, _E), jnp.float32),
    mesh=_mesh,
    scratch_types=[
        pltpu.VMEM((_NSB, _SB), jnp.int32),
        pltpu.VMEM((_RO + 16,), jnp.float32),
        pltpu.VMEM((_RO,), jnp.float32),
        pltpu.VMEM((_SB, _EC), jnp.float32),
        pltpu.VMEM((_SB, _EC), jnp.float32),
        pltpu.VMEM((_SB, _EC), jnp.float32),
        pltpu.VMEM_SHARED((_BW,), jnp.float32),
        pltpu.VMEM_SHARED((_BW, _EC), jnp.float32),
        pltpu.SemaphoreType.DMA,
        pltpu.SemaphoreType.DMA,
    ],
)
def _sc_pool(seg_hbm, emb_hbm, out_hbm, fid2, invc, small1d, buf0, buf1,
             ztab, cnt_sh, tab_sh, sem0, sem1):
    s = lax.axis_index("s")
    c = lax.axis_index("c")
    row0 = s * _RT
    bW = (s // 2) * _W
    own = s * _RO

    zero16 = jnp.zeros((16,), jnp.float32)
    one16 = jnp.ones((16,), jnp.float32)
    bufs = (buf0, buf1)
    sems = (sem0, sem1)

    def _fillz(r, carry):
        for j in range(_EC // 16):
            ztab[r, pl.ds(16 * j, 16)] = zero16
        return carry

    lax.fori_loop(0, _SB, _fillz, 0)

    def _fillz1(g, carry):
        small1d[pl.ds(16 * g, 16)] = zero16
        return carry

    lax.fori_loop(0, _RO // 16, _fillz1, 0)

    pltpu.sync_copy(seg_hbm.at[pl.ds(s * _NSB, _NSB)], fid2)

    def _addb(r, carry):
        for j in range(_SB // 16):
            fid2[r, pl.ds(16 * j, 16)] = fid2[r, pl.ds(16 * j, 16)] + bW
        return carry

    lax.fori_loop(0, _NSB, _addb, 0)

    pltpu.sync_copy(small1d.at[pl.ds(0, _RO)], cnt_sh.at[pl.ds(own, _RO)])
    for i in range(_RO // _SB):
        pltpu.sync_copy(ztab, tab_sh.at[pl.ds(own + i * _SB, _SB)])

    def _fillo(g, carry):
        small1d[pl.ds(16 * g, 16)] = one16
        return carry

    lax.fori_loop(0, _SB // 16, _fillo, 0)
    plsc.subcore_barrier()

    for sb in range(_NSB):
        pltpu.sync_copy(small1d.at[pl.ds(0, _SB)],
                        cnt_sh.at[fid2.at[sb]], add=True)
    plsc.subcore_barrier()

    pltpu.sync_copy(cnt_sh.at[pl.ds(own, _RO)], small1d.at[pl.ds(0, _RO)])

    def _binv(g, carry):
        v = small1d[pl.ds(16 * g, 16)]
        invc[pl.ds(16 * g, 16)] = 1.0 / jnp.maximum(v, 1.0)
        return carry

    lax.fori_loop(0, _RO // 16, _binv, 0)

    for k in range(_NCH):
        e0 = (c * _NCH + k) * _EC
        pending = pltpu.async_copy(
            emb_hbm.at[pl.ds(row0, _SB), pl.ds(e0, _EC)], bufs[0], sems[0])
        for sb in range(_NSB):
            pending.wait()
            if sb + 1 < _NSB:
                pending = pltpu.async_copy(
                    emb_hbm.at[pl.ds(row0 + (sb + 1) * _SB, _SB),
                               pl.ds(e0, _EC)],
                    bufs[(sb + 1) % 2], sems[(sb + 1) % 2])
            pltpu.sync_copy(bufs[sb % 2], tab_sh.at[fid2.at[sb]], add=True)
        plsc.subcore_barrier()
        for i in range(_RO // _SB):
            pltpu.sync_copy(tab_sh.at[pl.ds(own + i * _SB, _SB)], buf0)

            def _scale(r, carry, i=i):
                cs = invc[pl.ds(i * _SB + r, 16)][0]
                for j in range(_EC // 16):
                    buf0[r, pl.ds(16 * j, 16)] = (
                        buf0[r, pl.ds(16 * j, 16)] * cs)
                return carry

            lax.fori_loop(0, _SB, _scale, 0)
            pltpu.sync_copy(
                buf0, out_hbm.at[pl.ds(own + i * _SB, _SB), pl.ds(e0, _EC)])
        if k + 1 < _NCH:
            for i in range(_RO // _SB):
                pltpu.sync_copy(ztab, tab_sh.at[pl.ds(own + i * _SB, _SB)])
        plsc.subcore_barrier()


def kernel(subword_embs, segment_ids):
    seg2 = segment_ids.reshape(_SB, _SB).astype(jnp.int32)
    emb2 = subword_embs.reshape(_BL, _E)
    out = _sc_pool(seg2, emb2)
    return out.reshape(_B, _W, _E)


# speedup vs baseline: 1.4460x; 1.1206x over previous
import functools

import jax
import jax.numpy as jnp
from jax import lax
from jax.experimental import pallas as pl
from jax.experimental.pallas import tpu as pltpu
from jax.experimental.pallas import tpu_sc as plsc

_B, _L, _E, _W = 8, 2048, 768, 1024
_BL = _B * _L
_BW = _B * _W
_EC = 128
_NCH = 3
_RT = 1024
_SB = 128
_NSB = _RT // _SB
_RO = _BW // 16
_ZR = 64

_mesh = plsc.VectorSubcoreMesh(core_axis_name="c", subcore_axis_name="s")


@functools.partial(
    pl.kernel,
    out_type=jax.ShapeDtypeStruct((_BW, _E), jnp.float32),
    mesh=_mesh,
    scratch_types=[
        pltpu.VMEM((_NSB, _SB), jnp.int32),
        pltpu.VMEM((_RT + 16,), jnp.float32),
        pltpu.VMEM((_RT,), jnp.float32),
        pltpu.VMEM((_SB, _EC), jnp.float32),
        pltpu.VMEM((_SB, _EC), jnp.float32),
        pltpu.VMEM((_SB, _EC), jnp.float32),
        pltpu.VMEM((_ZR, _EC), jnp.float32),
        pltpu.VMEM_SHARED((_BW,), jnp.float32),
        pltpu.VMEM_SHARED((_BW, _EC), jnp.float32),
        pltpu.SemaphoreType.DMA,
        pltpu.SemaphoreType.DMA,
        pltpu.SemaphoreType.DMA,
        pltpu.SemaphoreType.DMA,
    ],
)
def _sc_pool(seg_hbm, emb_hbm, out_hbm, fid2, invs, small1d, buf0, buf1, buf2,
             ztab, cnt_sh, tab_sh, sem0, sem1, sem2, semw):
    s = lax.axis_index("s")
    c = lax.axis_index("c")
    row0 = s * _RT
    bW = (s // 2) * _W
    own = s * _RO

    zero16 = jnp.zeros((16,), jnp.float32)
    one16 = jnp.ones((16,), jnp.float32)
    bufs = (buf0, buf1, buf2)
    sems = (sem0, sem1, sem2)

    def _load(k, sb, j):
        e0 = (c * _NCH + k) * _EC
        return pltpu.async_copy(
            emb_hbm.at[pl.ds(row0 + sb * _SB, _SB), pl.ds(e0, _EC)],
            bufs[j], sems[j])

    ld = [_load(0, 0, 0), _load(0, 1, 1), None]

    def _fillz(r, carry):
        for j in range(_EC // 16):
            ztab[r, pl.ds(16 * j, 16)] = zero16
        return carry

    lax.fori_loop(0, _ZR, _fillz, 0)

    def _fillz1(g, carry):
        small1d[pl.ds(16 * g, 16)] = zero16
        return carry

    lax.fori_loop(0, _RO // 16, _fillz1, 0)

    pltpu.sync_copy(seg_hbm.at[pl.ds(s * _NSB, _NSB)], fid2)

    def _addb(r, carry):
        for j in range(_SB // 16):
            fid2[r, pl.ds(16 * j, 16)] = fid2[r, pl.ds(16 * j, 16)] + bW
        return carry

    lax.fori_loop(0, _NSB, _addb, 0)

    pltpu.sync_copy(small1d.at[pl.ds(0, _RO)], cnt_sh.at[pl.ds(own, _RO)])
    for i in range(_RO // _ZR):
        pltpu.sync_copy(ztab, tab_sh.at[pl.ds(own + i * _ZR, _ZR)])

    def _fillo(g, carry):
        small1d[pl.ds(16 * g, 16)] = one16
        return carry

    lax.fori_loop(0, _SB // 16, _fillo, 0)
    plsc.subcore_barrier()

    for sb in range(_NSB):
        pltpu.sync_copy(small1d.at[pl.ds(0, _SB)],
                        cnt_sh.at[fid2.at[sb]], add=True)
    plsc.subcore_barrier()

    for sb in range(_NSB):
        pltpu.sync_copy(cnt_sh.at[fid2.at[sb]],
                        small1d.at[pl.ds(sb * _SB, _SB)])

    def _binv(g, carry):
        v = small1d[pl.ds(16 * g, 16)]
        invs[pl.ds(16 * g, 16)] = 1.0 / v
        return carry

    lax.fori_loop(0, _RT // 16, _binv, 0)

    wout = None
    for k in range(_NCH):
        sc = [None, None, None]
        for sb in range(_NSB):
            j = sb % 3
            ld[j].wait()
            buf = bufs[j]

            def _scale(t, carry, sb=sb, buf=buf):
                for u in range(2):
                    r = 2 * t + u
                    cs = invs[pl.ds(sb * _SB + r, 16)][0]
                    for jj in range(_EC // 16):
                        buf[r, pl.ds(16 * jj, 16)] = (
                            buf[r, pl.ds(16 * jj, 16)] * cs)
                return carry

            lax.fori_loop(0, _SB // 2, _scale, 0)
            sc[j] = pltpu.async_copy(buf, tab_sh.at[fid2.at[sb]], sems[j],
                                     add=True)
            nxt = sb + 2
            if nxt < _NSB:
                jj = nxt % 3
                if sc[jj] is not None:
                    sc[jj].wait()
                ld[jj] = _load(k, nxt, jj)
        for j in (0, 1, 2):
            if sc[j] is not None:
                sc[j].wait()
        if k + 1 < _NCH:
            ld = [_load(k + 1, 0, 0), _load(k + 1, 1, 1), None]
        plsc.subcore_barrier()
        e0 = (c * _NCH + k) * _EC
        wout = pltpu.async_copy(
            tab_sh.at[pl.ds(own, _RO)],
            out_hbm.at[pl.ds(own, _RO), pl.ds(e0, _EC)], semw)
        if k + 1 < _NCH:
            wout.wait()
            zd = [pltpu.async_copy(ztab, tab_sh.at[pl.ds(own + i * _ZR, _ZR)],
                                   semw)
                  for i in range(_RO // _ZR)]
            for d in zd:
                d.wait()
            plsc.subcore_barrier()
    wout.wait()


def kernel(subword_embs, segment_ids):
    seg2 = segment_ids.reshape(_SB, _SB).astype(jnp.int32)
    emb2 = subword_embs.reshape(_BL, _E)
    out = _sc_pool(seg2, emb2)
    return out.reshape(_B, _W, _E)


# speedup vs baseline: 1.4605x; 1.0101x over previous
import functools

import jax
import jax.numpy as jnp
from jax import lax
from jax.experimental import pallas as pl
from jax.experimental.pallas import tpu as pltpu
from jax.experimental.pallas import tpu_sc as plsc

_B, _L, _E, _W = 8, 2048, 768, 1024
_BL = _B * _L
_BW = _B * _W
_EC = 128
_NCH = 3
_RT = 1024
_SB = 128
_NSB = _RT // _SB
_RO = _BW // 16
_ZR = 64

_mesh = plsc.VectorSubcoreMesh(core_axis_name="c", subcore_axis_name="s")


@functools.partial(
    pl.kernel,
    out_type=jax.ShapeDtypeStruct((_BW, _E), jnp.float32),
    mesh=_mesh,
    scratch_types=[
        pltpu.VMEM((_NSB, _SB), jnp.int32),
        pltpu.VMEM((_RT + 16,), jnp.float32),
        pltpu.VMEM((_RT,), jnp.float32),
        pltpu.VMEM((_SB, _EC), jnp.float32),
        pltpu.VMEM((_SB, _EC), jnp.float32),
        pltpu.VMEM((_SB, _EC), jnp.float32),
        pltpu.VMEM((_ZR, _EC), jnp.float32),
        pltpu.VMEM_SHARED((_BW,), jnp.float32),
        pltpu.VMEM_SHARED((_BW, _EC), jnp.float32),
        pltpu.SemaphoreType.DMA,
        pltpu.SemaphoreType.DMA,
        pltpu.SemaphoreType.DMA,
        pltpu.SemaphoreType.DMA,
    ],
)
def _sc_pool(seg_hbm, emb_hbm, out_hbm, fid2, invs, small1d, buf0, buf1, buf2,
             ztab, cnt_sh, tab_sh, sem0, sem1, sem2, semw):
    s = lax.axis_index("s")
    c = lax.axis_index("c")
    row0 = s * _RT
    bW = (s // 2) * _W
    own = s * _RO

    zero16 = jnp.zeros((16,), jnp.float32)
    one16 = jnp.ones((16,), jnp.float32)
    bufs = (buf0, buf1, buf2)
    sems = (sem0, sem1, sem2)

    def _load(k, sb, j):
        e0 = (c * _NCH + k) * _EC
        return pltpu.async_copy(
            emb_hbm.at[pl.ds(row0 + sb * _SB, _SB), pl.ds(e0, _EC)],
            bufs[j], sems[j])

    ld = [_load(0, 0, 0), _load(0, 1, 1), None]

    def _fillz(r, carry):
        for j in range(_EC // 16):
            ztab[r, pl.ds(16 * j, 16)] = zero16
        return carry

    lax.fori_loop(0, _ZR, _fillz, 0)

    def _fillz1(g, carry):
        small1d[pl.ds(16 * g, 16)] = zero16
        return carry

    lax.fori_loop(0, _RO // 16, _fillz1, 0)

    pltpu.sync_copy(seg_hbm.at[pl.ds(s * _NSB, _NSB)], fid2)

    def _addb(r, carry):
        for j in range(_SB // 16):
            fid2[r, pl.ds(16 * j, 16)] = fid2[r, pl.ds(16 * j, 16)] + bW
        return carry

    lax.fori_loop(0, _NSB, _addb, 0)

    pltpu.sync_copy(small1d.at[pl.ds(0, _RO)], cnt_sh.at[pl.ds(own, _RO)])
    for i in range(_RO // _ZR):
        pltpu.sync_copy(ztab, tab_sh.at[pl.ds(own + i * _ZR, _ZR)])

    def _fillo(g, carry):
        small1d[pl.ds(16 * g, 16)] = one16
        return carry

    lax.fori_loop(0, _SB // 16, _fillo, 0)
    plsc.subcore_barrier()

    cd = [pltpu.async_copy(small1d.at[pl.ds(0, _SB)],
                           cnt_sh.at[fid2.at[sb]], semw, add=True)
          for sb in range(_NSB)]
    for d in cd:
        d.wait()
    plsc.subcore_barrier()

    gd = [pltpu.async_copy(cnt_sh.at[fid2.at[sb]],
                           small1d.at[pl.ds(sb * _SB, _SB)], semw)
          for sb in range(_NSB)]
    for d in gd:
        d.wait()

    def _binv(g, carry):
        v = small1d[pl.ds(16 * g, 16)]
        invs[pl.ds(16 * g, 16)] = 1.0 / v
        return carry

    lax.fori_loop(0, _RT // 16, _binv, 0)

    wout = None
    for k in range(_NCH):
        sc = [None, None, None]
        for sb in range(_NSB):
            j = sb % 3
            ld[j].wait()
            buf = bufs[j]

            def _scale(t, carry, sb=sb, buf=buf):
                for u in range(2):
                    r = 2 * t + u
                    cs = invs[pl.ds(sb * _SB + r, 16)][0]
                    for jj in range(_EC // 16):
                        buf[r, pl.ds(16 * jj, 16)] = (
                            buf[r, pl.ds(16 * jj, 16)] * cs)
                return carry

            lax.fori_loop(0, _SB // 2, _scale, 0)
            sc[j] = pltpu.async_copy(buf, tab_sh.at[fid2.at[sb]], sems[j],
                                     add=True)
            nxt = sb + 2
            if nxt < _NSB:
                jj = nxt % 3
                if sc[jj] is not None:
                    sc[jj].wait()
                ld[jj] = _load(k, nxt, jj)
        for j in (0, 1, 2):
            if sc[j] is not None:
                sc[j].wait()
        if k + 1 < _NCH:
            ld = [_load(k + 1, 0, 0), _load(k + 1, 1, 1), None]
        plsc.subcore_barrier()
        e0 = (c * _NCH + k) * _EC
        wout = pltpu.async_copy(
            tab_sh.at[pl.ds(own, _RO)],
            out_hbm.at[pl.ds(own, _RO), pl.ds(e0, _EC)], semw)
        if k + 1 < _NCH:
            wout.wait()
            zd = [pltpu.async_copy(ztab, tab_sh.at[pl.ds(own + i * _ZR, _ZR)],
                                   semw)
                  for i in range(_RO // _ZR)]
            for d in zd:
                d.wait()
            plsc.subcore_barrier()
    wout.wait()


def kernel(subword_embs, segment_ids):
    seg2 = segment_ids.reshape(_SB, _SB).astype(jnp.int32)
    emb2 = subword_embs.reshape(_BL, _E)
    out = _sc_pool(seg2, emb2)
    return out.reshape(_B, _W, _E)
